# baseline (speedup 1.0000x reference)
import functools

import numpy as np
import jax
import jax.numpy as jnp
from jax import lax
from jax.experimental import pallas as pl
from jax.experimental.pallas import tpu as pltpu
from jax.experimental.pallas import tpu_sc as plsc

VOCAB = 1000000
D = 64
MAX_LEN = 512
PAD_IDX = 0
EPS = 1e-5


def _sinusoidal_pe(max_len, d):
    pos = np.arange(max_len)[:, None].astype(np.float32)
    div = np.exp(np.arange(0, d, 2).astype(np.float32) * (-np.log(10000.0) / d))
    pe = np.zeros((max_len, d), dtype=np.float32)
    pe[:, 0::2] = np.sin(pos * div)
    pe[:, 1::2] = np.cos(pos * div)
    return pe


@functools.lru_cache(maxsize=None)
def _make_sc_gather(n_tokens):
    info = plsc.get_sparse_core_info()
    nw = info.num_cores * info.num_subcores
    per_w = n_tokens // nw
    G = 128
    n_groups = per_w // G
    assert per_w % G == 0 and n_tokens % nw == 0
    mesh = plsc.VectorSubcoreMesh(core_axis_name="c", subcore_axis_name="s")

    @functools.partial(
        pl.kernel,
        mesh=mesh,
        compiler_params=pltpu.CompilerParams(use_tc_tiling_on_sc=False),
        out_type=jax.ShapeDtypeStruct((n_tokens, D), jnp.float32),
        scratch_types=[
            pltpu.VMEM((per_w,), jnp.int32),
            pltpu.VMEM((G, D), jnp.float32),
            pltpu.SemaphoreType.DMA,
        ],
    )
    def k(idx_hbm, table_hbm, out_hbm, idx_v, rows_v, sem):
        nc = info.num_cores
        wid = lax.axis_index("s") * nc + lax.axis_index("c")
        base = wid * per_w
        pltpu.sync_copy(idx_hbm.at[pl.ds(base, per_w)], idx_v)

        def body(j, carry):
            pltpu.async_copy(
                table_hbm.at[idx_v.at[pl.ds(j * G, G)]], rows_v, sem
            ).wait()
            pltpu.sync_copy(rows_v, out_hbm.at[pl.ds(base + j * G, G)])
            return carry

        lax.fori_loop(0, n_groups, body, 0)

    return k


def _ln_body(x_ref, emb_ref, pe_ref, gamma_ref, beta_ref, out_ref):
    emb = emb_ref[...]
    valid = x_ref[...] != PAD_IDX
    emb = jnp.where(valid, emb, 0.0)
    h = emb + pe_ref[...][None, :, :]
    mean = jnp.mean(h, axis=-1, keepdims=True)
    c = h - mean
    var = jnp.mean(c * c, axis=-1, keepdims=True)
    hn = c * lax.rsqrt(var + EPS)
    out_ref[...] = hn * gamma_ref[0][None, None, :] + beta_ref[0][None, None, :]


@functools.lru_cache(maxsize=None)
def _make_tc_ln(B, L, interpret=False):
    Bb = 16
    return pl.pallas_call(
        _ln_body,
        grid=(B // Bb,),
        in_specs=[
            pl.BlockSpec((Bb, L, 1), lambda i: (i, 0, 0)),
            pl.BlockSpec((Bb, L, D), lambda i: (i, 0, 0)),
            pl.BlockSpec((L, D), lambda i: (0, 0)),
            pl.BlockSpec((1, D), lambda i: (0, 0)),
            pl.BlockSpec((1, D), lambda i: (0, 0)),
        ],
        out_specs=pl.BlockSpec((Bb, L, D), lambda i: (i, 0, 0)),
        out_shape=jax.ShapeDtypeStruct((B, L, D), jnp.float32),
        interpret=interpret,
    )


def kernel(x, token_table, gamma, beta):
    B, L = x.shape
    idx_flat = x.reshape(-1)
    emb = _make_sc_gather(B * L)(idx_flat, token_table)
    emb = emb.reshape(B, L, D)
    pe = jnp.asarray(_sinusoidal_pe(MAX_LEN, D)[:L])
    return _make_tc_ln(B, L)(
        x.reshape(B, L, 1), emb, pe, gamma.reshape(1, D), beta.reshape(1, D)
    )
